# baseline (speedup 1.0000x reference)
import jax
import jax.numpy as jnp
from jax.experimental import pallas as pl
from jax.experimental.pallas import tpu as pltpu

T = 2048
D = 2048
E = 8
CAP = 2048
BT = 64
NBLK = T // BT


def _body(x_ref, w_ref, c_ref, comb_ref, idx_ref, loc_ref, la_ref, splits_ref,
          base_ref, me_ref):
    i = pl.program_id(0)

    @pl.when(i == 0)
    def _init():
        base_ref[...] = jnp.zeros((1, E), jnp.int32)
        me_ref[...] = jnp.zeros((1, E), jnp.float32)

    x = x_ref[...]
    w = w_ref[...]
    c = c_ref[...]

    xr = jax.lax.dot_general(x, w, (((1,), (1,)), ((), ())),
                             preferred_element_type=jnp.float32)
    n1 = jnp.sqrt(jnp.sum(c * c, axis=1, keepdims=True))
    c2 = c * (1.5 / n1)
    n2 = jnp.sqrt(jnp.sum(c2 * c2, axis=1, keepdims=True))
    cn = c2 / jnp.maximum(n2, 1e-4)
    logits = jax.lax.dot_general(xr, cn, (((1,), (1,)), ((), ())),
                                 preferred_element_type=jnp.float32)

    m = jnp.max(logits, axis=1, keepdims=True)
    ex = jnp.exp(logits - m)
    s = jnp.sum(ex, axis=1, keepdims=True)
    gates = ex / s
    gate1 = 1.5 / s

    iota_e = jax.lax.broadcasted_iota(jnp.int32, (BT, E), 1)
    idx = jnp.min(jnp.where(logits == m, iota_e, E), axis=1, keepdims=True)
    mask_f = (iota_e == idx).astype(jnp.float32)

    me_ref[...] = me_ref[...] + jnp.sum(gates, axis=0, keepdims=True)
    cnt = jnp.sum(mask_f, axis=0, keepdims=True)

    r_io = jax.lax.broadcasted_iota(jnp.int32, (BT, BT), 0)
    c_io = jax.lax.broadcasted_iota(jnp.int32, (BT, BT), 1)
    tri = (r_io > c_io).astype(jnp.float32)
    prior = jax.lax.dot_general(tri, mask_f, (((1,), (0,)), ((), ())),
                                preferred_element_type=jnp.float32)
    base_f = base_ref[...].astype(jnp.float32)
    locf = jnp.sum(mask_f * (prior + base_f), axis=1, keepdims=True)
    loc = locf.astype(jnp.int32)
    base_ref[...] = base_ref[...] + cnt.astype(jnp.int32)

    idx_ref[...] = idx
    loc_ref[...] = loc

    e_io = jax.lax.broadcasted_iota(jnp.int32, (BT, E, CAP), 1)
    c3_io = jax.lax.broadcasted_iota(jnp.int32, (BT, E, CAP), 2)
    hit = jnp.logical_and(e_io == idx[:, :, None], c3_io == loc[:, :, None])
    comb_ref[...] = jnp.where(hit, gate1[:, :, None], 0.0)

    @pl.when(i == NBLK - 1)
    def _fin():
        counts = base_ref[...].astype(jnp.float32)
        me = me_ref[...] * (1.0 / T)
        ce = counts * (1.0 / T)
        prod = jnp.sum(me * ce, axis=1, keepdims=True) * float(E)
        la_ref[...] = prod
        splits_ref[...] = base_ref[...]


def kernel(input, W, expert_centroids):
    comb, idxs, locs, la, splits = pl.pallas_call(
        _body,
        grid=(NBLK,),
        in_specs=[
            pl.BlockSpec((BT, D), lambda i: (i, 0)),
            pl.BlockSpec((4, D), lambda i: (0, 0)),
            pl.BlockSpec((E, 4), lambda i: (0, 0)),
        ],
        out_specs=[
            pl.BlockSpec((BT, E, CAP), lambda i: (i, 0, 0)),
            pl.BlockSpec((BT, 1), lambda i: (i, 0)),
            pl.BlockSpec((BT, 1), lambda i: (i, 0)),
            pl.BlockSpec((1, 1), lambda i: (0, 0)),
            pl.BlockSpec((1, E), lambda i: (0, 0)),
        ],
        out_shape=[
            jax.ShapeDtypeStruct((T, E, CAP), jnp.float32),
            jax.ShapeDtypeStruct((T, 1), jnp.int32),
            jax.ShapeDtypeStruct((T, 1), jnp.int32),
            jax.ShapeDtypeStruct((1, 1), jnp.float32),
            jax.ShapeDtypeStruct((1, E), jnp.int32),
        ],
        scratch_shapes=[
            pltpu.VMEM((1, E), jnp.int32),
            pltpu.VMEM((1, E), jnp.float32),
        ],
        compiler_params=pltpu.CompilerParams(
            dimension_semantics=("arbitrary",),
        ),
    )(input, W, expert_centroids)

    oh_e = idxs == jnp.arange(E, dtype=jnp.int32)[None, :]
    oh_c = locs == jnp.arange(CAP, dtype=jnp.int32)[None, :]
    disp = jnp.logical_and(oh_e[:, :, None], oh_c[:, None, :])
    return (la.reshape(()), comb, disp, splits.reshape(E))


# speedup vs baseline: 1.0538x; 1.0538x over previous
import jax
import jax.numpy as jnp
from jax.experimental import pallas as pl
from jax.experimental.pallas import tpu as pltpu

T = 2048
D = 2048
E = 8
CAP = 2048
BR = 256
NR = T // BR
BT = 128
NBLK = T // BT


def _route_body(x_ref, w_ref, c_ref, idx_ref, loc_ref, gate_ref, la_ref,
                splits_ref, base_ref, me_ref):
    i = pl.program_id(0)

    @pl.when(i == 0)
    def _init():
        base_ref[...] = jnp.zeros((1, E), jnp.int32)
        me_ref[...] = jnp.zeros((1, E), jnp.float32)

    x = x_ref[...]
    w = w_ref[...]
    c = c_ref[...]

    xr = jax.lax.dot_general(x, w, (((1,), (1,)), ((), ())),
                             preferred_element_type=jnp.float32)
    n1 = jnp.sqrt(jnp.sum(c * c, axis=1, keepdims=True))
    c2 = c * (1.5 / n1)
    n2 = jnp.sqrt(jnp.sum(c2 * c2, axis=1, keepdims=True))
    cn = c2 / jnp.maximum(n2, 1e-4)
    logits = jax.lax.dot_general(xr, cn, (((1,), (1,)), ((), ())),
                                 preferred_element_type=jnp.float32)

    m = jnp.max(logits, axis=1, keepdims=True)
    ex = jnp.exp(logits - m)
    s = jnp.sum(ex, axis=1, keepdims=True)
    gates = ex / s

    iota_e = jax.lax.broadcasted_iota(jnp.int32, (BR, E), 1)
    idx = jnp.min(jnp.where(logits == m, iota_e, E), axis=1, keepdims=True)
    mask_f = (iota_e == idx).astype(jnp.float32)

    me_ref[...] = me_ref[...] + jnp.sum(gates, axis=0, keepdims=True)
    cnt = jnp.sum(mask_f, axis=0, keepdims=True)

    r_io = jax.lax.broadcasted_iota(jnp.int32, (BR, BR), 0)
    c_io = jax.lax.broadcasted_iota(jnp.int32, (BR, BR), 1)
    tri = (r_io > c_io).astype(jnp.float32)
    prior = jax.lax.dot_general(tri, mask_f, (((1,), (0,)), ((), ())),
                                preferred_element_type=jnp.float32)
    base_f = base_ref[...].astype(jnp.float32)
    locf = jnp.sum(mask_f * (prior + base_f), axis=1, keepdims=True)
    base_ref[...] = base_ref[...] + cnt.astype(jnp.int32)

    idx_ref[...] = idx
    loc_ref[...] = locf.astype(jnp.int32)
    gate_ref[...] = 1.5 / s

    @pl.when(i == NR - 1)
    def _fin():
        counts = base_ref[...].astype(jnp.float32)
        me = me_ref[...] * (1.0 / T)
        ce = counts * (1.0 / T)
        la_ref[...] = jnp.sum(me * ce, axis=1, keepdims=True) * float(E)
        splits_ref[...] = base_ref[...]


def _fill_body(idx_ref, loc_ref, gate_ref, comb_ref):
    idx = idx_ref[...]
    loc = loc_ref[...]
    gate = gate_ref[...]
    e_io = jax.lax.broadcasted_iota(jnp.int32, (BT, E, CAP), 1)
    c_io = jax.lax.broadcasted_iota(jnp.int32, (BT, E, CAP), 2)
    hit = jnp.logical_and(e_io == idx[:, :, None], c_io == loc[:, :, None])
    comb_ref[...] = jnp.where(hit, gate[:, :, None], 0.0)


def kernel(input, W, expert_centroids):
    idxs, locs, gate1, la, splits = pl.pallas_call(
        _route_body,
        grid=(NR,),
        in_specs=[
            pl.BlockSpec((BR, D), lambda i: (i, 0)),
            pl.BlockSpec((4, D), lambda i: (0, 0)),
            pl.BlockSpec((E, 4), lambda i: (0, 0)),
        ],
        out_specs=[
            pl.BlockSpec((BR, 1), lambda i: (i, 0)),
            pl.BlockSpec((BR, 1), lambda i: (i, 0)),
            pl.BlockSpec((BR, 1), lambda i: (i, 0)),
            pl.BlockSpec((1, 1), lambda i: (0, 0)),
            pl.BlockSpec((1, E), lambda i: (0, 0)),
        ],
        out_shape=[
            jax.ShapeDtypeStruct((T, 1), jnp.int32),
            jax.ShapeDtypeStruct((T, 1), jnp.int32),
            jax.ShapeDtypeStruct((T, 1), jnp.float32),
            jax.ShapeDtypeStruct((1, 1), jnp.float32),
            jax.ShapeDtypeStruct((1, E), jnp.int32),
        ],
        scratch_shapes=[
            pltpu.VMEM((1, E), jnp.int32),
            pltpu.VMEM((1, E), jnp.float32),
        ],
        compiler_params=pltpu.CompilerParams(
            dimension_semantics=("arbitrary",),
        ),
    )(input, W, expert_centroids)

    comb = pl.pallas_call(
        _fill_body,
        grid=(NBLK,),
        in_specs=[
            pl.BlockSpec((BT, 1), lambda i: (i, 0)),
            pl.BlockSpec((BT, 1), lambda i: (i, 0)),
            pl.BlockSpec((BT, 1), lambda i: (i, 0)),
        ],
        out_specs=[
            pl.BlockSpec((BT, E, CAP), lambda i: (i, 0, 0)),
        ],
        out_shape=[
            jax.ShapeDtypeStruct((T, E, CAP), jnp.float32),
        ],
        compiler_params=pltpu.CompilerParams(
            dimension_semantics=("arbitrary",),
        ),
    )(idxs, locs, gate1)[0]

    oh_e = idxs == jnp.arange(E, dtype=jnp.int32)[None, :]
    oh_c = locs == jnp.arange(CAP, dtype=jnp.int32)[None, :]
    disp = jnp.logical_and(oh_e[:, :, None], oh_c[:, None, :])
    return (la.reshape(()), comb, disp, splits.reshape(E))


# speedup vs baseline: 1.1465x; 1.0879x over previous
import jax
import jax.numpy as jnp
from jax.experimental import pallas as pl
from jax.experimental.pallas import tpu as pltpu

T = 2048
D = 2048
E = 8
CAP = 2048
BT = 128
NBLK = T // BT


def _body(x_ref, w_ref, c_ref, comb_ref, idx_ref, loc_ref, la_ref, splits_ref,
          base_ref, me_ref):
    i = pl.program_id(0)

    @pl.when(i == 0)
    def _init():
        base_ref[...] = jnp.zeros((1, E), jnp.int32)
        me_ref[...] = jnp.zeros((1, E), jnp.float32)

    x = x_ref[...]
    w = w_ref[...]
    c = c_ref[...]

    xr = jax.lax.dot_general(x, w, (((1,), (1,)), ((), ())),
                             preferred_element_type=jnp.float32)
    n1 = jnp.sqrt(jnp.sum(c * c, axis=1, keepdims=True))
    c2 = c * (1.5 / n1)
    n2 = jnp.sqrt(jnp.sum(c2 * c2, axis=1, keepdims=True))
    cn = c2 / jnp.maximum(n2, 1e-4)
    logits = jax.lax.dot_general(xr, cn, (((1,), (1,)), ((), ())),
                                 preferred_element_type=jnp.float32)

    m = jnp.max(logits, axis=1, keepdims=True)
    ex = jnp.exp(logits - m)
    s = jnp.sum(ex, axis=1, keepdims=True)
    gates = ex / s
    gate1 = 1.5 / s

    iota_e = jax.lax.broadcasted_iota(jnp.int32, (BT, E), 1)
    idx = jnp.min(jnp.where(logits == m, iota_e, E), axis=1, keepdims=True)
    mask_f = (iota_e == idx).astype(jnp.float32)

    me_ref[...] = me_ref[...] + jnp.sum(gates, axis=0, keepdims=True)
    cnt = jnp.sum(mask_f, axis=0, keepdims=True)

    r_io = jax.lax.broadcasted_iota(jnp.int32, (BT, BT), 0)
    c_io = jax.lax.broadcasted_iota(jnp.int32, (BT, BT), 1)
    tri = (r_io > c_io).astype(jnp.float32)
    prior = jax.lax.dot_general(tri, mask_f, (((1,), (0,)), ((), ())),
                                preferred_element_type=jnp.float32)
    base_f = base_ref[...].astype(jnp.float32)
    locf = jnp.sum(mask_f * (prior + base_f), axis=1, keepdims=True)
    loc = locf.astype(jnp.int32)
    base_ref[...] = base_ref[...] + cnt.astype(jnp.int32)

    idx_ref[...] = idx
    loc_ref[...] = loc

    e_io = jax.lax.broadcasted_iota(jnp.int32, (BT, E, CAP), 1)
    c3_io = jax.lax.broadcasted_iota(jnp.int32, (BT, E, CAP), 2)
    hit = jnp.logical_and(e_io == idx[:, :, None], c3_io == loc[:, :, None])
    comb_ref[...] = jnp.where(hit, gate1[:, :, None], 0.0)

    @pl.when(i == NBLK - 1)
    def _fin():
        counts = base_ref[...].astype(jnp.float32)
        me = me_ref[...] * (1.0 / T)
        ce = counts * (1.0 / T)
        prod = jnp.sum(me * ce, axis=1, keepdims=True) * float(E)
        la_ref[...] = prod
        splits_ref[...] = base_ref[...]


def kernel(input, W, expert_centroids):
    comb, idxs, locs, la, splits = pl.pallas_call(
        _body,
        grid=(NBLK,),
        in_specs=[
            pl.BlockSpec((BT, D), lambda i: (i, 0)),
            pl.BlockSpec((4, D), lambda i: (0, 0)),
            pl.BlockSpec((E, 4), lambda i: (0, 0)),
        ],
        out_specs=[
            pl.BlockSpec((BT, E, CAP), lambda i: (i, 0, 0)),
            pl.BlockSpec((BT, 1), lambda i: (i, 0)),
            pl.BlockSpec((BT, 1), lambda i: (i, 0)),
            pl.BlockSpec((1, 1), lambda i: (0, 0)),
            pl.BlockSpec((1, E), lambda i: (0, 0)),
        ],
        out_shape=[
            jax.ShapeDtypeStruct((T, E, CAP), jnp.float32),
            jax.ShapeDtypeStruct((T, 1), jnp.int32),
            jax.ShapeDtypeStruct((T, 1), jnp.int32),
            jax.ShapeDtypeStruct((1, 1), jnp.float32),
            jax.ShapeDtypeStruct((1, E), jnp.int32),
        ],
        scratch_shapes=[
            pltpu.VMEM((1, E), jnp.int32),
            pltpu.VMEM((1, E), jnp.float32),
        ],
        compiler_params=pltpu.CompilerParams(
            dimension_semantics=("arbitrary",),
        ),
    )(input, W, expert_centroids)

    oh_e = idxs == jnp.arange(E, dtype=jnp.int32)[None, :]
    oh_c = locs == jnp.arange(CAP, dtype=jnp.int32)[None, :]
    disp = jnp.logical_and(oh_e[:, :, None], oh_c[:, None, :])
    return (la.reshape(()), comb, disp, splits.reshape(E))
